# baseline (speedup 1.0000x reference)
import jax
import jax.numpy as jnp
from jax.experimental import pallas as pl
from jax.experimental.pallas import tpu as pltpu

D_MODEL = 2048
NUM_EXPERTS = 64
TOP_K = 2
SCALE = 2.0
D_HALF = D_MODEL // 2
N_TOKENS = 8192

BLK = 1024
GRID = N_TOKENS // BLK


def _router_body(ux_ref, uy_ref, ex_ref, ey_ref, bias_ref, ab_ref, cd_ref,
                 ti_ref, ts_ref, sc_ref, aux_ref, acc_ref):
    i = pl.program_id(0)

    px = jax.lax.dot(ux_ref[...], ex_ref[...],
                     preferred_element_type=jnp.float32)
    py = jax.lax.dot(uy_ref[...], ey_ref[...],
                     preferred_element_type=jnp.float32)
    pre = jnp.concatenate([px, py], axis=1)
    ta = jnp.abs(jnp.tanh(pre) * SCALE)
    la = jnp.log(ta)
    powa = jnp.exp(ab_ref[...] * la)
    gauss = jnp.exp(-jnp.exp(cd_ref[...] * la))
    s = ((powa[:, :NUM_EXPERTS] + powa[:, NUM_EXPERTS:])
         * (gauss[:, :NUM_EXPERTS] * gauss[:, NUM_EXPERTS:])
         + bias_ref[...])
    sc_ref[...] = s

    cols = jax.lax.broadcasted_iota(jnp.int32, s.shape, 1)
    m1 = jnp.max(s, axis=1, keepdims=True)
    i1 = jnp.min(jnp.where(s == m1, cols, NUM_EXPERTS), axis=1, keepdims=True)
    masked = jnp.where(cols == i1, -jnp.inf, s)
    m2 = jnp.max(masked, axis=1, keepdims=True)
    i2 = jnp.min(jnp.where(masked == m2, cols, NUM_EXPERTS), axis=1,
                 keepdims=True)
    ts_ref[...] = jnp.concatenate([m1, m2], axis=1)
    ti_ref[...] = jnp.concatenate([i1, i2], axis=1)

    e = jnp.exp(s - m1)
    ones_n = jnp.ones((NUM_EXPERTS, 1), jnp.float32)
    srow = jax.lax.dot(e, ones_n, preferred_element_type=jnp.float32)
    recip = (1.0 / srow).reshape(1, BLK)
    psum = jax.lax.dot(recip, e, preferred_element_type=jnp.float32)

    @pl.when(i == 0)
    def _():
        acc_ref[...] = jnp.zeros_like(acc_ref)

    acc_ref[...] += psum

    @pl.when(i == GRID - 1)
    def _():
        mean = acc_ref[...] * (1.0 / N_TOKENS)
        aux_ref[...] = jnp.sum(mean * mean, keepdims=True) * NUM_EXPERTS


def kernel(u, E_x, E_y, bias, a1, b1, c, d):
    bias2 = jnp.reshape(bias, (1, NUM_EXPERTS))
    ab = jnp.concatenate([jnp.full((1, NUM_EXPERTS), a1, jnp.float32),
                          jnp.full((1, NUM_EXPERTS), b1, jnp.float32)], axis=1)
    cd = jnp.concatenate([jnp.full((1, NUM_EXPERTS), c, jnp.float32),
                          jnp.full((1, NUM_EXPERTS), d, jnp.float32)], axis=1)

    topk_i, topk_s, scores, aux = pl.pallas_call(
        _router_body,
        grid=(GRID,),
        in_specs=[
            pl.BlockSpec((BLK, D_HALF), lambda i: (i, 0)),
            pl.BlockSpec((BLK, D_HALF), lambda i: (i, 1)),
            pl.BlockSpec((D_HALF, NUM_EXPERTS), lambda i: (0, 0)),
            pl.BlockSpec((D_HALF, NUM_EXPERTS), lambda i: (0, 0)),
            pl.BlockSpec((1, NUM_EXPERTS), lambda i: (0, 0)),
            pl.BlockSpec((1, 2 * NUM_EXPERTS), lambda i: (0, 0)),
            pl.BlockSpec((1, 2 * NUM_EXPERTS), lambda i: (0, 0)),
        ],
        out_specs=[
            pl.BlockSpec((BLK, TOP_K), lambda i: (i, 0)),
            pl.BlockSpec((BLK, TOP_K), lambda i: (i, 0)),
            pl.BlockSpec((BLK, NUM_EXPERTS), lambda i: (i, 0)),
            pl.BlockSpec((1, 1), lambda i: (0, 0)),
        ],
        out_shape=[
            jax.ShapeDtypeStruct((N_TOKENS, TOP_K), jnp.int32),
            jax.ShapeDtypeStruct((N_TOKENS, TOP_K), jnp.float32),
            jax.ShapeDtypeStruct((N_TOKENS, NUM_EXPERTS), jnp.float32),
            jax.ShapeDtypeStruct((1, 1), jnp.float32),
        ],
        scratch_shapes=[pltpu.VMEM((1, NUM_EXPERTS), jnp.float32)],
    )(u, u, E_x, E_y, bias2, ab, cd)

    return (topk_i, topk_s, scores, aux[0, 0])


# speedup vs baseline: 1.1121x; 1.1121x over previous
import jax
import jax.numpy as jnp
from jax.experimental import pallas as pl
from jax.experimental.pallas import tpu as pltpu

D_MODEL = 2048
NUM_EXPERTS = 64
TOP_K = 2
SCALE = 2.0
D_HALF = D_MODEL // 2
N_TOKENS = 8192

BLK = 1024
GRID = N_TOKENS // BLK


def _router_body(ux_ref, uy_ref, ex_ref, ey_ref, bias_ref, ab_ref, cd_ref,
                 ti_ref, ts_ref, sc_ref, aux_ref, acc_ref):
    i = pl.program_id(0)

    px = jax.lax.dot(ux_ref[...], ex_ref[...],
                     preferred_element_type=jnp.float32)
    py = jax.lax.dot(uy_ref[...], ey_ref[...],
                     preferred_element_type=jnp.float32)
    pre = jnp.concatenate([px, py], axis=1)
    ta = jnp.abs(jnp.tanh(pre) * SCALE)
    la = jnp.log(ta)
    powa = jnp.exp(ab_ref[...] * la)
    gauss = jnp.exp(-jnp.exp(cd_ref[...] * la))
    s = ((powa[:, :NUM_EXPERTS] + powa[:, NUM_EXPERTS:])
         * (gauss[:, :NUM_EXPERTS] * gauss[:, NUM_EXPERTS:])
         + bias_ref[...])
    sc_ref[...] = s

    cols = jax.lax.broadcasted_iota(jnp.int32, s.shape, 1)
    m1 = jnp.max(s, axis=1, keepdims=True)
    i1 = jnp.min(jnp.where(s == m1, cols, NUM_EXPERTS), axis=1, keepdims=True)
    masked = jnp.where(cols == i1, -jnp.inf, s)
    m2 = jnp.max(masked, axis=1, keepdims=True)
    i2 = jnp.min(jnp.where(masked == m2, cols, NUM_EXPERTS), axis=1,
                 keepdims=True)
    ts_ref[...] = jnp.concatenate([m1, m2], axis=1)
    ti_ref[...] = jnp.concatenate([i1, i2], axis=1)

    e = jnp.exp(s - m1)
    p = e * (1.0 / jnp.sum(e, axis=1, keepdims=True))
    psum = jnp.sum(p, axis=0, keepdims=True)

    @pl.when(i == 0)
    def _():
        acc_ref[...] = jnp.zeros_like(acc_ref)

    acc_ref[...] += psum

    @pl.when(i == GRID - 1)
    def _():
        mean = acc_ref[...] * (1.0 / N_TOKENS)
        aux_ref[...] = jnp.sum(mean * mean, keepdims=True) * NUM_EXPERTS


def kernel(u, E_x, E_y, bias, a1, b1, c, d):
    bias2 = jnp.reshape(bias, (1, NUM_EXPERTS))
    ab = jnp.concatenate([jnp.full((1, NUM_EXPERTS), a1, jnp.float32),
                          jnp.full((1, NUM_EXPERTS), b1, jnp.float32)], axis=1)
    cd = jnp.concatenate([jnp.full((1, NUM_EXPERTS), c, jnp.float32),
                          jnp.full((1, NUM_EXPERTS), d, jnp.float32)], axis=1)

    topk_i, topk_s, scores, aux = pl.pallas_call(
        _router_body,
        grid=(GRID,),
        in_specs=[
            pl.BlockSpec((BLK, D_HALF), lambda i: (i, 0)),
            pl.BlockSpec((BLK, D_HALF), lambda i: (i, 1)),
            pl.BlockSpec((D_HALF, NUM_EXPERTS), lambda i: (0, 0)),
            pl.BlockSpec((D_HALF, NUM_EXPERTS), lambda i: (0, 0)),
            pl.BlockSpec((1, NUM_EXPERTS), lambda i: (0, 0)),
            pl.BlockSpec((1, 2 * NUM_EXPERTS), lambda i: (0, 0)),
            pl.BlockSpec((1, 2 * NUM_EXPERTS), lambda i: (0, 0)),
        ],
        out_specs=[
            pl.BlockSpec((BLK, TOP_K), lambda i: (i, 0)),
            pl.BlockSpec((BLK, TOP_K), lambda i: (i, 0)),
            pl.BlockSpec((BLK, NUM_EXPERTS), lambda i: (i, 0)),
            pl.BlockSpec((1, 1), lambda i: (0, 0)),
        ],
        out_shape=[
            jax.ShapeDtypeStruct((N_TOKENS, TOP_K), jnp.int32),
            jax.ShapeDtypeStruct((N_TOKENS, TOP_K), jnp.float32),
            jax.ShapeDtypeStruct((N_TOKENS, NUM_EXPERTS), jnp.float32),
            jax.ShapeDtypeStruct((1, 1), jnp.float32),
        ],
        scratch_shapes=[pltpu.VMEM((1, NUM_EXPERTS), jnp.float32)],
    )(u, u, E_x, E_y, bias2, ab, cd)

    return (topk_i, topk_s, scores, aux[0, 0])


# speedup vs baseline: 1.1585x; 1.0417x over previous
import jax
import jax.numpy as jnp
from jax.experimental import pallas as pl
from jax.experimental.pallas import tpu as pltpu

D_MODEL = 2048
NUM_EXPERTS = 64
TOP_K = 2
SCALE = 2.0
D_HALF = D_MODEL // 2
N_TOKENS = 8192

BLK = 1024
GRID = N_TOKENS // BLK


def _router_body(u_hbm, ex_ref, ey_ref, bias_ref, scal_ref,
                 ti_ref, ts_ref, sc_ref, aux_ref, buf_ref, acc_ref, sem):
    i = pl.program_id(0)
    cur = jax.lax.rem(i, 2)
    nxt = jax.lax.rem(i + 1, 2)

    @pl.when(i == 0)
    def _():
        pltpu.make_async_copy(u_hbm.at[pl.ds(0, BLK)],
                              buf_ref.at[0], sem.at[0]).start()

    @pl.when(i + 1 < GRID)
    def _():
        pltpu.make_async_copy(u_hbm.at[pl.ds((i + 1) * BLK, BLK)],
                              buf_ref.at[nxt], sem.at[nxt]).start()

    pltpu.make_async_copy(u_hbm.at[pl.ds(i * BLK, BLK)],
                          buf_ref.at[cur], sem.at[cur]).wait()
    ub = buf_ref[cur]

    x = jnp.tanh(jax.lax.dot(ub[:, :D_HALF], ex_ref[...],
                             preferred_element_type=jnp.float32)) * SCALE
    y = jnp.tanh(jax.lax.dot(ub[:, D_HALF:], ey_ref[...],
                             preferred_element_type=jnp.float32)) * SCALE

    a1 = scal_ref[0, 0]
    b1 = scal_ref[0, 1]
    c = scal_ref[0, 2]
    d = scal_ref[0, 3]
    xa = jnp.abs(x)
    ya = jnp.abs(y)
    s = (xa ** a1 + ya ** b1) * jnp.exp(-(xa ** c + ya ** d)) + bias_ref[...]
    sc_ref[...] = s

    cols = jax.lax.broadcasted_iota(jnp.int32, s.shape, 1)
    m1 = jnp.max(s, axis=1, keepdims=True)
    i1 = jnp.min(jnp.where(s == m1, cols, NUM_EXPERTS), axis=1, keepdims=True)
    masked = jnp.where(cols == i1, -jnp.inf, s)
    m2 = jnp.max(masked, axis=1, keepdims=True)
    i2 = jnp.min(jnp.where(masked == m2, cols, NUM_EXPERTS), axis=1,
                 keepdims=True)
    ts_ref[...] = jnp.concatenate([m1, m2], axis=1)
    ti_ref[...] = jnp.concatenate([i1, i2], axis=1)

    e = jnp.exp(s - m1)
    p = e * (1.0 / jnp.sum(e, axis=1, keepdims=True))
    psum = jnp.sum(p, axis=0, keepdims=True)

    @pl.when(i == 0)
    def _():
        acc_ref[...] = jnp.zeros_like(acc_ref)

    acc_ref[...] += psum

    @pl.when(i == GRID - 1)
    def _():
        mean = acc_ref[...] * (1.0 / N_TOKENS)
        aux_ref[...] = jnp.sum(mean * mean, keepdims=True) * NUM_EXPERTS


def kernel(u, E_x, E_y, bias, a1, b1, c, d):
    bias2 = jnp.reshape(bias, (1, NUM_EXPERTS))
    scal = jnp.stack([jnp.asarray(a1, jnp.float32), jnp.asarray(b1, jnp.float32),
                      jnp.asarray(c, jnp.float32), jnp.asarray(d, jnp.float32)]
                     ).reshape(1, 4)

    topk_i, topk_s, scores, aux = pl.pallas_call(
        _router_body,
        grid=(GRID,),
        in_specs=[
            pl.BlockSpec(memory_space=pl.ANY),
            pl.BlockSpec((D_HALF, NUM_EXPERTS), lambda i: (0, 0)),
            pl.BlockSpec((D_HALF, NUM_EXPERTS), lambda i: (0, 0)),
            pl.BlockSpec((1, NUM_EXPERTS), lambda i: (0, 0)),
            pl.BlockSpec(memory_space=pltpu.SMEM),
        ],
        out_specs=[
            pl.BlockSpec((BLK, TOP_K), lambda i: (i, 0)),
            pl.BlockSpec((BLK, TOP_K), lambda i: (i, 0)),
            pl.BlockSpec((BLK, NUM_EXPERTS), lambda i: (i, 0)),
            pl.BlockSpec((1, 1), lambda i: (0, 0)),
        ],
        out_shape=[
            jax.ShapeDtypeStruct((N_TOKENS, TOP_K), jnp.int32),
            jax.ShapeDtypeStruct((N_TOKENS, TOP_K), jnp.float32),
            jax.ShapeDtypeStruct((N_TOKENS, NUM_EXPERTS), jnp.float32),
            jax.ShapeDtypeStruct((1, 1), jnp.float32),
        ],
        scratch_shapes=[pltpu.VMEM((2, BLK, D_MODEL), jnp.float32),
                        pltpu.VMEM((1, NUM_EXPERTS), jnp.float32),
                        pltpu.SemaphoreType.DMA((2,))],
    )(u, E_x, E_y, bias2, scal)

    return (topk_i, topk_s, scores, aux[0, 0])


# speedup vs baseline: 1.3636x; 1.1771x over previous
import jax
import jax.numpy as jnp
from jax.experimental import pallas as pl
from jax.experimental.pallas import tpu as pltpu

D_MODEL = 2048
NUM_EXPERTS = 64
TOP_K = 2
SCALE = 2.0
D_HALF = D_MODEL // 2
N_TOKENS = 8192

BLK = 1024
GRID = N_TOKENS // BLK


def _router_body(ux_ref, uy_ref, ex_ref, ey_ref, biasT_ref, scal_ref,
                 ti_ref, ts_ref, sc_ref, aux_ref, acc_ref):
    i = pl.program_id(0)

    x = jax.lax.dot(ux_ref[...], ex_ref[...],
                    preferred_element_type=jnp.float32)
    y = jax.lax.dot(uy_ref[...], ey_ref[...],
                    preferred_element_type=jnp.float32)
    xT = x.T
    yT = y.T

    a1 = scal_ref[0, 0]
    b1 = scal_ref[0, 1]
    c = scal_ref[0, 2]
    d = scal_ref[0, 3]
    lx = jnp.log(jnp.abs(jnp.tanh(xT) * SCALE))
    ly = jnp.log(jnp.abs(jnp.tanh(yT) * SCALE))
    sT = ((jnp.exp(a1 * lx) + jnp.exp(b1 * ly))
          * jnp.exp(-jnp.exp(c * lx) - jnp.exp(d * ly)) + biasT_ref[...])
    sc_ref[...] = sT.T

    rows = jax.lax.broadcasted_iota(jnp.int32, sT.shape, 0)
    m1 = jnp.max(sT, axis=0, keepdims=True)
    i1 = jnp.min(jnp.where(sT == m1, rows, NUM_EXPERTS), axis=0,
                 keepdims=True)
    masked = jnp.where(rows == i1, -jnp.inf, sT)
    m2 = jnp.max(masked, axis=0, keepdims=True)
    i2 = jnp.min(jnp.where(masked == m2, rows, NUM_EXPERTS), axis=0,
                 keepdims=True)
    ts_ref[...] = jnp.concatenate([m1, m2], axis=0).T
    ti_ref[...] = jnp.concatenate([i1, i2], axis=0).T

    e = jnp.exp(sT - m1)
    p = e * (1.0 / jnp.sum(e, axis=0, keepdims=True))
    psum = jnp.sum(p, axis=1, keepdims=True)

    @pl.when(i == 0)
    def _():
        acc_ref[...] = jnp.zeros_like(acc_ref)

    acc_ref[...] += psum

    @pl.when(i == GRID - 1)
    def _():
        mean = acc_ref[...] * (1.0 / N_TOKENS)
        aux_ref[...] = jnp.sum(mean * mean, keepdims=True) * NUM_EXPERTS


def kernel(u, E_x, E_y, bias, a1, b1, c, d):
    biasT = jnp.reshape(bias, (NUM_EXPERTS, 1))
    scal = jnp.stack([jnp.asarray(a1, jnp.float32), jnp.asarray(b1, jnp.float32),
                      jnp.asarray(c, jnp.float32), jnp.asarray(d, jnp.float32)]
                     ).reshape(1, 4)

    topk_i, topk_s, scores, aux = pl.pallas_call(
        _router_body,
        grid=(GRID,),
        in_specs=[
            pl.BlockSpec((BLK, D_HALF), lambda i: (i, 0)),
            pl.BlockSpec((BLK, D_HALF), lambda i: (i, 1)),
            pl.BlockSpec((D_HALF, NUM_EXPERTS), lambda i: (0, 0)),
            pl.BlockSpec((D_HALF, NUM_EXPERTS), lambda i: (0, 0)),
            pl.BlockSpec((NUM_EXPERTS, 1), lambda i: (0, 0)),
            pl.BlockSpec(memory_space=pltpu.SMEM),
        ],
        out_specs=[
            pl.BlockSpec((BLK, TOP_K), lambda i: (i, 0)),
            pl.BlockSpec((BLK, TOP_K), lambda i: (i, 0)),
            pl.BlockSpec((BLK, NUM_EXPERTS), lambda i: (i, 0)),
            pl.BlockSpec((1, 1), lambda i: (0, 0)),
        ],
        out_shape=[
            jax.ShapeDtypeStruct((N_TOKENS, TOP_K), jnp.int32),
            jax.ShapeDtypeStruct((N_TOKENS, TOP_K), jnp.float32),
            jax.ShapeDtypeStruct((N_TOKENS, NUM_EXPERTS), jnp.float32),
            jax.ShapeDtypeStruct((1, 1), jnp.float32),
        ],
        scratch_shapes=[pltpu.VMEM((NUM_EXPERTS, 1), jnp.float32)],
    )(u, u, E_x, E_y, biasT, scal)

    return (topk_i, topk_s, scores, aux[0, 0])


# speedup vs baseline: 1.3716x; 1.0058x over previous
import jax
import jax.numpy as jnp
from jax.experimental import pallas as pl
from jax.experimental.pallas import tpu as pltpu

D_MODEL = 2048
NUM_EXPERTS = 64
TOP_K = 2
SCALE = 2.0
D_HALF = D_MODEL // 2
N_TOKENS = 8192

BLK = 2048
GRID = N_TOKENS // BLK


def _router_body(ux_ref, uy_ref, ex_ref, ey_ref, biasT_ref, scal_ref,
                 ti_ref, ts_ref, sc_ref, aux_ref, acc_ref):
    i = pl.program_id(0)

    x = jax.lax.dot(ux_ref[...], ex_ref[...],
                    preferred_element_type=jnp.float32)
    y = jax.lax.dot(uy_ref[...], ey_ref[...],
                    preferred_element_type=jnp.float32)
    xT = x.T
    yT = y.T

    a1 = scal_ref[0, 0]
    b1 = scal_ref[0, 1]
    c = scal_ref[0, 2]
    d = scal_ref[0, 3]
    lx = jnp.log(jnp.abs(jnp.tanh(xT) * SCALE))
    ly = jnp.log(jnp.abs(jnp.tanh(yT) * SCALE))
    sT = ((jnp.exp(a1 * lx) + jnp.exp(b1 * ly))
          * jnp.exp(-jnp.exp(c * lx) - jnp.exp(d * ly)) + biasT_ref[...])
    sc_ref[...] = sT.T

    rows = jax.lax.broadcasted_iota(jnp.int32, sT.shape, 0)
    m1 = jnp.max(sT, axis=0, keepdims=True)
    i1 = jnp.min(jnp.where(sT == m1, rows, NUM_EXPERTS), axis=0,
                 keepdims=True)
    masked = jnp.where(rows == i1, -jnp.inf, sT)
    m2 = jnp.max(masked, axis=0, keepdims=True)
    i2 = jnp.min(jnp.where(masked == m2, rows, NUM_EXPERTS), axis=0,
                 keepdims=True)
    ts_ref[...] = jnp.concatenate([m1, m2], axis=0).T
    ti_ref[...] = jnp.concatenate([i1, i2], axis=0).T

    e = jnp.exp(sT - m1)
    p = e * (1.0 / jnp.sum(e, axis=0, keepdims=True))
    psum = jnp.sum(p, axis=1, keepdims=True)

    @pl.when(i == 0)
    def _():
        acc_ref[...] = jnp.zeros_like(acc_ref)

    acc_ref[...] += psum

    @pl.when(i == GRID - 1)
    def _():
        mean = acc_ref[...] * (1.0 / N_TOKENS)
        aux_ref[...] = jnp.sum(mean * mean, keepdims=True) * NUM_EXPERTS


def kernel(u, E_x, E_y, bias, a1, b1, c, d):
    biasT = jnp.reshape(bias, (NUM_EXPERTS, 1))
    scal = jnp.stack([jnp.asarray(a1, jnp.float32), jnp.asarray(b1, jnp.float32),
                      jnp.asarray(c, jnp.float32), jnp.asarray(d, jnp.float32)]
                     ).reshape(1, 4)

    topk_i, topk_s, scores, aux = pl.pallas_call(
        _router_body,
        grid=(GRID,),
        in_specs=[
            pl.BlockSpec((BLK, D_HALF), lambda i: (i, 0)),
            pl.BlockSpec((BLK, D_HALF), lambda i: (i, 1)),
            pl.BlockSpec((D_HALF, NUM_EXPERTS), lambda i: (0, 0)),
            pl.BlockSpec((D_HALF, NUM_EXPERTS), lambda i: (0, 0)),
            pl.BlockSpec((NUM_EXPERTS, 1), lambda i: (0, 0)),
            pl.BlockSpec(memory_space=pltpu.SMEM),
        ],
        out_specs=[
            pl.BlockSpec((BLK, TOP_K), lambda i: (i, 0)),
            pl.BlockSpec((BLK, TOP_K), lambda i: (i, 0)),
            pl.BlockSpec((BLK, NUM_EXPERTS), lambda i: (i, 0)),
            pl.BlockSpec((1, 1), lambda i: (0, 0)),
        ],
        out_shape=[
            jax.ShapeDtypeStruct((N_TOKENS, TOP_K), jnp.int32),
            jax.ShapeDtypeStruct((N_TOKENS, TOP_K), jnp.float32),
            jax.ShapeDtypeStruct((N_TOKENS, NUM_EXPERTS), jnp.float32),
            jax.ShapeDtypeStruct((1, 1), jnp.float32),
        ],
        scratch_shapes=[pltpu.VMEM((NUM_EXPERTS, 1), jnp.float32)],
    )(u, u, E_x, E_y, biasT, scal)

    return (topk_i, topk_s, scores, aux[0, 0])


# speedup vs baseline: 1.3742x; 1.0019x over previous
import jax
import jax.numpy as jnp
from jax.experimental import pallas as pl
from jax.experimental.pallas import tpu as pltpu

D_MODEL = 2048
NUM_EXPERTS = 64
TOP_K = 2
SCALE = 2.0
D_HALF = D_MODEL // 2
N_TOKENS = 8192

BLK = 2048
GRID = N_TOKENS // BLK


def _router_body(ux_ref, uy_ref, ex_ref, ey_ref, biasT_ref, scal_ref,
                 ti_ref, ts_ref, sc_ref, aux_ref, acc_ref):
    i = pl.program_id(0)

    dn = (((0,), (1,)), ((), ()))
    xT = jax.lax.dot_general(ex_ref[...], ux_ref[...], dn,
                             preferred_element_type=jnp.float32)
    yT = jax.lax.dot_general(ey_ref[...], uy_ref[...], dn,
                             preferred_element_type=jnp.float32)

    a1 = scal_ref[0, 0]
    b1 = scal_ref[0, 1]
    c = scal_ref[0, 2]
    d = scal_ref[0, 3]
    lx = jnp.log(jnp.abs(jnp.tanh(xT) * SCALE))
    ly = jnp.log(jnp.abs(jnp.tanh(yT) * SCALE))
    sT = ((jnp.exp(a1 * lx) + jnp.exp(b1 * ly))
          * jnp.exp(-jnp.exp(c * lx) - jnp.exp(d * ly)) + biasT_ref[...])
    sc_ref[...] = sT.T

    rows = jax.lax.broadcasted_iota(jnp.int32, sT.shape, 0)
    m1 = jnp.max(sT, axis=0, keepdims=True)
    i1 = jnp.min(jnp.where(sT == m1, rows, NUM_EXPERTS), axis=0,
                 keepdims=True)
    masked = jnp.where(rows == i1, -jnp.inf, sT)
    m2 = jnp.max(masked, axis=0, keepdims=True)
    i2 = jnp.min(jnp.where(masked == m2, rows, NUM_EXPERTS), axis=0,
                 keepdims=True)
    ts_ref[...] = jnp.concatenate([m1, m2], axis=0).T
    ti_ref[...] = jnp.concatenate([i1, i2], axis=0).T

    e = jnp.exp(sT - m1)
    p = e * (1.0 / jnp.sum(e, axis=0, keepdims=True))
    psum = jnp.sum(p, axis=1, keepdims=True)

    @pl.when(i == 0)
    def _():
        acc_ref[...] = jnp.zeros_like(acc_ref)

    acc_ref[...] += psum

    @pl.when(i == GRID - 1)
    def _():
        mean = acc_ref[...] * (1.0 / N_TOKENS)
        aux_ref[...] = jnp.sum(mean * mean, keepdims=True) * NUM_EXPERTS


def kernel(u, E_x, E_y, bias, a1, b1, c, d):
    biasT = jnp.reshape(bias, (NUM_EXPERTS, 1))
    scal = jnp.stack([jnp.asarray(a1, jnp.float32), jnp.asarray(b1, jnp.float32),
                      jnp.asarray(c, jnp.float32), jnp.asarray(d, jnp.float32)]
                     ).reshape(1, 4)

    topk_i, topk_s, scores, aux = pl.pallas_call(
        _router_body,
        grid=(GRID,),
        in_specs=[
            pl.BlockSpec((BLK, D_HALF), lambda i: (i, 0)),
            pl.BlockSpec((BLK, D_HALF), lambda i: (i, 1)),
            pl.BlockSpec((D_HALF, NUM_EXPERTS), lambda i: (0, 0)),
            pl.BlockSpec((D_HALF, NUM_EXPERTS), lambda i: (0, 0)),
            pl.BlockSpec((NUM_EXPERTS, 1), lambda i: (0, 0)),
            pl.BlockSpec(memory_space=pltpu.SMEM),
        ],
        out_specs=[
            pl.BlockSpec((BLK, TOP_K), lambda i: (i, 0)),
            pl.BlockSpec((BLK, TOP_K), lambda i: (i, 0)),
            pl.BlockSpec((BLK, NUM_EXPERTS), lambda i: (i, 0)),
            pl.BlockSpec((1, 1), lambda i: (0, 0)),
        ],
        out_shape=[
            jax.ShapeDtypeStruct((N_TOKENS, TOP_K), jnp.int32),
            jax.ShapeDtypeStruct((N_TOKENS, TOP_K), jnp.float32),
            jax.ShapeDtypeStruct((N_TOKENS, NUM_EXPERTS), jnp.float32),
            jax.ShapeDtypeStruct((1, 1), jnp.float32),
        ],
        scratch_shapes=[pltpu.VMEM((NUM_EXPERTS, 1), jnp.float32)],
    )(u, u, E_x, E_y, biasT, scal)

    return (topk_i, topk_s, scores, aux[0, 0])
